# baseline (device time: 25633 ns/iter reference)
import jax
import jax.numpy as jnp
from jax import lax
from jax.experimental import pallas as pl
from jax.experimental.pallas import tpu as pltpu

CHUNK_ROWS = (256, 256, 192, 128, 96, 48, 32, 16)
K = len(CHUNK_ROWS)
CHUNK_OFFS = tuple(sum(CHUNK_ROWS[:i]) for i in range(K))


def kernel(x):
    m, n = x.shape
    nh = n // 2
    mh = m // 2
    out_m = 2 * m
    assert sum(CHUNK_ROWS) == mh

    def body(x_ref, out_ref, send_buf, recv_buf, local_buf,
             xsend_sems, xrecv_sems, ysend_sems, yrecv_sems, local_sem):
        my_x = lax.axis_index("x")
        my_y = lax.axis_index("y")
        my_z = lax.axis_index("z")
        px = 1 - my_x
        ny = 1 - my_y

        barrier_sem = pltpu.get_barrier_semaphore()
        pl.semaphore_signal(
            barrier_sem, inc=1, device_id=(px, my_y, my_z),
            device_id_type=pl.DeviceIdType.MESH,
        )
        pl.semaphore_signal(
            barrier_sem, inc=1, device_id=(my_x, ny, my_z),
            device_id_type=pl.DeviceIdType.MESH,
        )
        pl.semaphore_wait(barrier_sem, 2)

        send_base = my_x * m + my_y * mh
        recv_base = px * m + my_y * mh

        x_rdmas = []
        for i in range(K):
            send_buf[pl.ds(CHUNK_OFFS[i], CHUNK_ROWS[i]), :] = x_ref[
                pl.ds(my_y * mh + CHUNK_OFFS[i], CHUNK_ROWS[i]),
                pl.ds(px * nh, nh),
            ].astype(jnp.bfloat16)
            r = pltpu.make_async_remote_copy(
                src_ref=send_buf.at[pl.ds(CHUNK_OFFS[i], CHUNK_ROWS[i]), :],
                dst_ref=recv_buf.at[pl.ds(CHUNK_OFFS[i], CHUNK_ROWS[i]), :],
                send_sem=xsend_sems.at[i],
                recv_sem=xrecv_sems.at[i],
                device_id=(px, my_y, my_z),
                device_id_type=pl.DeviceIdType.MESH,
            )
            r.start()
            x_rdmas.append(r)

        local_buf[...] = x_ref[:, pl.ds(my_x * nh, nh)].astype(jnp.bfloat16)
        local_cp = pltpu.make_async_copy(
            local_buf, out_ref.at[pl.ds(my_x * m, m), :], local_sem
        )
        local_cp.start()

        y_rdmas = []
        recv_cps = []
        for i in range(K):
            x_rdmas[i].wait_recv()
            r = pltpu.make_async_remote_copy(
                src_ref=recv_buf.at[pl.ds(CHUNK_OFFS[i], CHUNK_ROWS[i]), :],
                dst_ref=out_ref.at[
                    pl.ds(recv_base + CHUNK_OFFS[i], CHUNK_ROWS[i]), :
                ],
                send_sem=ysend_sems.at[i],
                recv_sem=yrecv_sems.at[i],
                device_id=(my_x, ny, my_z),
                device_id_type=pl.DeviceIdType.MESH,
            )
            r.start()
            y_rdmas.append(r)
            cp = pltpu.make_async_copy(
                recv_buf.at[pl.ds(CHUNK_OFFS[i], CHUNK_ROWS[i]), :],
                out_ref.at[pl.ds(recv_base + CHUNK_OFFS[i], CHUNK_ROWS[i]), :],
                local_sem,
            )
            cp.start()
            recv_cps.append(cp)

        local_cp.wait()
        for i in range(K):
            recv_cps[i].wait()
            y_rdmas[i].wait_recv()
            x_rdmas[i].wait_send()
            y_rdmas[i].wait_send()

    return pl.pallas_call(
        body,
        out_shape=jax.ShapeDtypeStruct((out_m, nh), jnp.bfloat16),
        in_specs=[pl.BlockSpec(memory_space=pltpu.VMEM)],
        out_specs=pl.BlockSpec(memory_space=pltpu.MemorySpace.HBM),
        scratch_shapes=[
            pltpu.VMEM((mh, nh), jnp.bfloat16),
            pltpu.VMEM((mh, nh), jnp.bfloat16),
            pltpu.VMEM((m, nh), jnp.bfloat16),
            pltpu.SemaphoreType.DMA((K,)),
            pltpu.SemaphoreType.DMA((K,)),
            pltpu.SemaphoreType.DMA((K,)),
            pltpu.SemaphoreType.DMA((K,)),
            pltpu.SemaphoreType.DMA,
        ],
        compiler_params=pltpu.CompilerParams(collective_id=0),
    )(x)


# device time: 23183 ns/iter; 1.1057x vs baseline; 1.1057x over previous
import jax
import jax.numpy as jnp
from jax import lax
from jax.experimental import pallas as pl
from jax.experimental.pallas import tpu as pltpu

CK = 128
XK = 9
FK = 7


def kernel(x):
    m, n = x.shape
    nh = n // 2
    mh = m // 2
    out_m = 2 * m
    tail = FK * CK
    assert mh == 8 * CK

    def body(x_ref, out_ref, send_buf, xsend_sems, xrecv_sems,
             ysend_sems, yrecv_sems):
        my_x = lax.axis_index("x")
        my_y = lax.axis_index("y")
        my_z = lax.axis_index("z")
        px = 1 - my_x
        ny = 1 - my_y

        barrier_sem = pltpu.get_barrier_semaphore()
        pl.semaphore_signal(
            barrier_sem, inc=1, device_id=(px, my_y, my_z),
            device_id_type=pl.DeviceIdType.MESH,
        )
        pl.semaphore_signal(
            barrier_sem, inc=1, device_id=(my_x, ny, my_z),
            device_id_type=pl.DeviceIdType.MESH,
        )
        pl.semaphore_wait(barrier_sem, 2)

        send_buf[pl.ds(0, mh), :] = x_ref[
            pl.ds(my_y * mh, mh), pl.ds(px * nh, nh)
        ].astype(jnp.bfloat16)
        send_buf[pl.ds(mh, CK), :] = x_ref[
            pl.ds(ny * mh + tail, CK), pl.ds(px * nh, nh)
        ].astype(jnp.bfloat16)

        send_base = my_x * m + my_y * mh
        recv_base = px * m + my_y * mh
        send_tail_dst = my_x * m + ny * mh + tail

        x_rdmas = []
        for i in range(XK):
            if i < 8:
                dst = out_ref.at[pl.ds(send_base + i * CK, CK), :]
            else:
                dst = out_ref.at[pl.ds(send_tail_dst, CK), :]
            r = pltpu.make_async_remote_copy(
                src_ref=send_buf.at[pl.ds(i * CK, CK), :],
                dst_ref=dst,
                send_sem=xsend_sems.at[i],
                recv_sem=xrecv_sems.at[i],
                device_id=(px, my_y, my_z),
                device_id_type=pl.DeviceIdType.MESH,
            )
            r.start()
            x_rdmas.append(r)

        out_ref[pl.ds(my_x * m, m), :] = x_ref[
            :, pl.ds(my_x * nh, nh)
        ].astype(jnp.bfloat16)

        y_rdmas = []
        for i in range(FK):
            x_rdmas[i].wait_recv()
            r = pltpu.make_async_remote_copy(
                src_ref=out_ref.at[pl.ds(recv_base + i * CK, CK), :],
                dst_ref=out_ref.at[pl.ds(recv_base + i * CK, CK), :],
                send_sem=ysend_sems.at[i],
                recv_sem=yrecv_sems.at[i],
                device_id=(my_x, ny, my_z),
                device_id_type=pl.DeviceIdType.MESH,
            )
            r.start()
            y_rdmas.append(r)

        for i in range(FK, XK):
            x_rdmas[i].wait_recv()
        for i in range(FK):
            y_rdmas[i].wait_recv()
            y_rdmas[i].wait_send()
        for i in range(XK):
            x_rdmas[i].wait_send()

    return pl.pallas_call(
        body,
        out_shape=jax.ShapeDtypeStruct((out_m, nh), jnp.bfloat16),
        in_specs=[pl.BlockSpec(memory_space=pltpu.MemorySpace.VMEM)],
        out_specs=pl.BlockSpec(memory_space=pltpu.MemorySpace.VMEM),
        scratch_shapes=[
            pltpu.VMEM((XK * CK, nh), jnp.bfloat16),
            pltpu.SemaphoreType.DMA((XK,)),
            pltpu.SemaphoreType.DMA((XK,)),
            pltpu.SemaphoreType.DMA((FK,)),
            pltpu.SemaphoreType.DMA((FK,)),
        ],
        compiler_params=pltpu.CompilerParams(collective_id=0),
    )(x)


# device time: 22306 ns/iter; 1.1492x vs baseline; 1.0393x over previous
import jax
import jax.numpy as jnp
from jax import lax
from jax.experimental import pallas as pl
from jax.experimental.pallas import tpu as pltpu

CK = 128
QK = 4


def kernel(x):
    m, n = x.shape
    nh = n // 2
    qm = QK * CK
    out_m = 2 * m
    assert m == 4 * qm

    def body(x_ref, out_ref, send_buf,
             xsend_sems, xrecv_sems, ysend_sems, yrecv_sems,
             zsend_sems, zrecv_sems):
        my_x = lax.axis_index("x")
        my_y = lax.axis_index("y")
        my_z = lax.axis_index("z")
        px = 1 - my_x
        ny = 1 - my_y
        j = lax.rem(my_z, 2)
        nz = my_z + 1 - 2 * j

        xdev = (px, my_y, my_z)
        ydev = (my_x, ny, my_z)
        zdev = (my_x, my_y, nz)

        barrier_sem = pltpu.get_barrier_semaphore()
        for dev in (xdev, ydev, zdev):
            pl.semaphore_signal(
                barrier_sem, inc=1, device_id=dev,
                device_id_type=pl.DeviceIdType.MESH,
            )
        pl.semaphore_wait(barrier_sem, 3)

        my_q = 2 * my_y + j
        diag_q = 2 * ny + (1 - j)
        yq = 2 * ny + j
        zq = 2 * my_y + (1 - j)

        send_buf[pl.ds(0, qm), :] = x_ref[
            pl.ds(my_q * qm, qm), pl.ds(px * nh, nh)
        ].astype(jnp.bfloat16)
        extra_row = diag_q * qm + 2 * j * CK
        send_buf[pl.ds(qm, 2 * CK), :] = x_ref[
            pl.ds(extra_row, 2 * CK), pl.ds(px * nh, nh)
        ].astype(jnp.bfloat16)

        send_block = my_x * m
        recv_block = px * m

        def remote(src, dst_row, send_sem, recv_sem, dev):
            r = pltpu.make_async_remote_copy(
                src_ref=src,
                dst_ref=out_ref.at[pl.ds(dst_row, CK), :],
                send_sem=send_sem,
                recv_sem=recv_sem,
                device_id=dev,
                device_id_type=pl.DeviceIdType.MESH,
            )
            r.start()
            return r

        x_rdmas = []
        for i in range(6):
            if i < QK:
                dst_row = send_block + my_q * qm + i * CK
            else:
                dst_row = send_block + extra_row + (i - QK) * CK
            x_rdmas.append(remote(
                send_buf.at[pl.ds(i * CK, CK), :], dst_row,
                xsend_sems.at[i], xrecv_sems.at[i], xdev,
            ))

        out_ref[pl.ds(my_x * m, m), :] = x_ref[
            :, pl.ds(my_x * nh, nh)
        ].astype(jnp.bfloat16)

        y_rdmas = []
        z_rdmas = []
        for i in range(QK):
            x_rdmas[i].wait_recv()
            row = recv_block + my_q * qm + i * CK
            src = out_ref.at[pl.ds(row, CK), :]
            y_rdmas.append(remote(
                src, row, ysend_sems.at[i], yrecv_sems.at[i], ydev))
            z_rdmas.append(remote(
                src, row, zsend_sems.at[i], zrecv_sems.at[i], zdev))

        ky = 2 * (1 - j)
        z_rdmas_wait_slot_ky = pltpu.make_async_remote_copy(
            src_ref=send_buf.at[pl.ds(0, CK), :],
            dst_ref=out_ref.at[pl.ds(recv_block + zq * qm + ky * CK, CK), :],
            send_sem=zsend_sems.at[4],
            recv_sem=zrecv_sems.at[ky],
            device_id=zdev,
            device_id_type=pl.DeviceIdType.MESH,
        )
        z_rdmas_wait_slot_ky.wait_recv()
        row_y_relay = recv_block + zq * qm + ky * CK
        y_rdmas.append(remote(
            out_ref.at[pl.ds(row_y_relay, CK), :], row_y_relay,
            ysend_sems.at[4], yrecv_sems.at[4], ydev))

        kz = 2 * j + 1
        y_rdmas_wait_slot_kz = pltpu.make_async_remote_copy(
            src_ref=send_buf.at[pl.ds(0, CK), :],
            dst_ref=out_ref.at[pl.ds(recv_block + yq * qm + kz * CK, CK), :],
            send_sem=ysend_sems.at[4],
            recv_sem=yrecv_sems.at[kz],
            device_id=ydev,
            device_id_type=pl.DeviceIdType.MESH,
        )
        y_rdmas_wait_slot_kz.wait_recv()
        row_z_relay = recv_block + yq * qm + kz * CK
        z_rdmas.append(remote(
            out_ref.at[pl.ds(row_z_relay, CK), :], row_z_relay,
            zsend_sems.at[4], zrecv_sems.at[4], zdev))

        for i in range(QK, 6):
            x_rdmas[i].wait_recv()
        for i in range(QK):
            @pl.when(i != kz)
            def _():
                pltpu.make_async_remote_copy(
                    src_ref=send_buf.at[pl.ds(0, CK), :],
                    dst_ref=out_ref.at[pl.ds(recv_block + yq * qm + i * CK, CK), :],
                    send_sem=ysend_sems.at[4],
                    recv_sem=yrecv_sems.at[i],
                    device_id=ydev,
                    device_id_type=pl.DeviceIdType.MESH,
                ).wait_recv()
        pltpu.make_async_remote_copy(
            src_ref=send_buf.at[pl.ds(0, CK), :],
            dst_ref=out_ref.at[pl.ds(recv_block + diag_q * qm + ky * CK, CK), :],
            send_sem=ysend_sems.at[4],
            recv_sem=yrecv_sems.at[4],
            device_id=ydev,
            device_id_type=pl.DeviceIdType.MESH,
        ).wait_recv()
        for i in range(QK):
            @pl.when(i != ky)
            def _():
                pltpu.make_async_remote_copy(
                    src_ref=send_buf.at[pl.ds(0, CK), :],
                    dst_ref=out_ref.at[pl.ds(recv_block + zq * qm + i * CK, CK), :],
                    send_sem=zsend_sems.at[4],
                    recv_sem=zrecv_sems.at[i],
                    device_id=zdev,
                    device_id_type=pl.DeviceIdType.MESH,
                ).wait_recv()
        pltpu.make_async_remote_copy(
            src_ref=send_buf.at[pl.ds(0, CK), :],
            dst_ref=out_ref.at[
                pl.ds(recv_block + diag_q * qm + (2 * (1 - j) + 1) * CK, CK), :
            ],
            send_sem=zsend_sems.at[4],
            recv_sem=zrecv_sems.at[4],
            device_id=zdev,
            device_id_type=pl.DeviceIdType.MESH,
        ).wait_recv()
        for r in x_rdmas:
            r.wait_send()
        for r in y_rdmas:
            r.wait_send()
        for r in z_rdmas:
            r.wait_send()

    return pl.pallas_call(
        body,
        out_shape=jax.ShapeDtypeStruct((out_m, nh), jnp.bfloat16),
        in_specs=[pl.BlockSpec(memory_space=pltpu.MemorySpace.VMEM)],
        out_specs=pl.BlockSpec(memory_space=pltpu.MemorySpace.VMEM),
        scratch_shapes=[
            pltpu.VMEM((6 * CK, nh), jnp.bfloat16),
            pltpu.SemaphoreType.DMA((6,)),
            pltpu.SemaphoreType.DMA((6,)),
            pltpu.SemaphoreType.DMA((5,)),
            pltpu.SemaphoreType.DMA((5,)),
            pltpu.SemaphoreType.DMA((5,)),
            pltpu.SemaphoreType.DMA((5,)),
        ],
        compiler_params=pltpu.CompilerParams(collective_id=0),
    )(x)


# device time: 22277 ns/iter; 1.1506x vs baseline; 1.0013x over previous
import jax
import jax.numpy as jnp
from jax import lax
from jax.experimental import pallas as pl
from jax.experimental.pallas import tpu as pltpu

CK = 128
QK = 4


def kernel(x):
    m, n = x.shape
    nh = n // 2
    qm = QK * CK
    out_m = 2 * m
    assert m == 4 * qm

    def body(x_ref, out_ref, send_buf,
             xsend_sems, xrecv_sems, ysend_sems, yrecv_sems,
             zsend_sems, zrecv_sems):
        my_x = lax.axis_index("x")
        my_y = lax.axis_index("y")
        my_z = lax.axis_index("z")
        px = 1 - my_x
        ny = 1 - my_y
        j = lax.rem(my_z, 2)
        nz = my_z + 1 - 2 * j

        xdev = (px, my_y, my_z)
        ydev = (my_x, ny, my_z)
        zdev = (my_x, my_y, nz)

        barrier_sem = pltpu.get_barrier_semaphore()
        for dev in (xdev, ydev, zdev):
            pl.semaphore_signal(
                barrier_sem, inc=1, device_id=dev,
                device_id_type=pl.DeviceIdType.MESH,
            )
        pl.semaphore_wait(barrier_sem, 3)

        my_q = 2 * my_y + j
        diag_q = 2 * ny + (1 - j)
        yq = 2 * ny + j
        zq = 2 * my_y + (1 - j)

        extra_row = diag_q * qm + 2 * j * CK

        send_block = my_x * m
        recv_block = px * m

        def remote(src, dst_row, send_sem, recv_sem, dev):
            r = pltpu.make_async_remote_copy(
                src_ref=src,
                dst_ref=out_ref.at[pl.ds(dst_row, CK), :],
                send_sem=send_sem,
                recv_sem=recv_sem,
                device_id=dev,
                device_id_type=pl.DeviceIdType.MESH,
            )
            r.start()
            return r

        send_buf[pl.ds(0, qm), :] = x_ref[
            pl.ds(my_q * qm, qm), pl.ds(px * nh, nh)
        ].astype(jnp.bfloat16)
        x_rdmas = []
        for i in range(QK):
            x_rdmas.append(remote(
                send_buf.at[pl.ds(i * CK, CK), :],
                send_block + my_q * qm + i * CK,
                xsend_sems.at[i], xrecv_sems.at[i], xdev,
            ))
        send_buf[pl.ds(qm, 2 * CK), :] = x_ref[
            pl.ds(extra_row, 2 * CK), pl.ds(px * nh, nh)
        ].astype(jnp.bfloat16)
        for i in range(QK, 6):
            x_rdmas.append(remote(
                send_buf.at[pl.ds(i * CK, CK), :],
                send_block + extra_row + (i - QK) * CK,
                xsend_sems.at[i], xrecv_sems.at[i], xdev,
            ))

        y_rdmas = []
        z_rdmas = []
        for i in range(QK):
            x_rdmas[i].wait_recv()
            row = recv_block + my_q * qm + i * CK
            src = out_ref.at[pl.ds(row, CK), :]
            y_rdmas.append(remote(
                src, row, ysend_sems.at[i], yrecv_sems.at[i], ydev))
            z_rdmas.append(remote(
                src, row, zsend_sems.at[i], zrecv_sems.at[i], zdev))

        ky = 2 * (1 - j)
        z_rdmas_wait_slot_ky = pltpu.make_async_remote_copy(
            src_ref=send_buf.at[pl.ds(0, CK), :],
            dst_ref=out_ref.at[pl.ds(recv_block + zq * qm + ky * CK, CK), :],
            send_sem=zsend_sems.at[4],
            recv_sem=zrecv_sems.at[ky],
            device_id=zdev,
            device_id_type=pl.DeviceIdType.MESH,
        )
        z_rdmas_wait_slot_ky.wait_recv()
        row_y_relay = recv_block + zq * qm + ky * CK
        y_rdmas.append(remote(
            out_ref.at[pl.ds(row_y_relay, CK), :], row_y_relay,
            ysend_sems.at[4], yrecv_sems.at[4], ydev))

        kz = 2 * j + 1
        y_rdmas_wait_slot_kz = pltpu.make_async_remote_copy(
            src_ref=send_buf.at[pl.ds(0, CK), :],
            dst_ref=out_ref.at[pl.ds(recv_block + yq * qm + kz * CK, CK), :],
            send_sem=ysend_sems.at[4],
            recv_sem=yrecv_sems.at[kz],
            device_id=ydev,
            device_id_type=pl.DeviceIdType.MESH,
        )
        y_rdmas_wait_slot_kz.wait_recv()
        row_z_relay = recv_block + yq * qm + kz * CK
        z_rdmas.append(remote(
            out_ref.at[pl.ds(row_z_relay, CK), :], row_z_relay,
            zsend_sems.at[4], zrecv_sems.at[4], zdev))

        out_ref[pl.ds(my_x * m, m), :] = x_ref[
            :, pl.ds(my_x * nh, nh)
        ].astype(jnp.bfloat16)

        for i in range(QK, 6):
            x_rdmas[i].wait_recv()
        for i in range(QK):
            @pl.when(i != kz)
            def _():
                pltpu.make_async_remote_copy(
                    src_ref=send_buf.at[pl.ds(0, CK), :],
                    dst_ref=out_ref.at[pl.ds(recv_block + yq * qm + i * CK, CK), :],
                    send_sem=ysend_sems.at[4],
                    recv_sem=yrecv_sems.at[i],
                    device_id=ydev,
                    device_id_type=pl.DeviceIdType.MESH,
                ).wait_recv()
        pltpu.make_async_remote_copy(
            src_ref=send_buf.at[pl.ds(0, CK), :],
            dst_ref=out_ref.at[pl.ds(recv_block + diag_q * qm + ky * CK, CK), :],
            send_sem=ysend_sems.at[4],
            recv_sem=yrecv_sems.at[4],
            device_id=ydev,
            device_id_type=pl.DeviceIdType.MESH,
        ).wait_recv()
        for i in range(QK):
            @pl.when(i != ky)
            def _():
                pltpu.make_async_remote_copy(
                    src_ref=send_buf.at[pl.ds(0, CK), :],
                    dst_ref=out_ref.at[pl.ds(recv_block + zq * qm + i * CK, CK), :],
                    send_sem=zsend_sems.at[4],
                    recv_sem=zrecv_sems.at[i],
                    device_id=zdev,
                    device_id_type=pl.DeviceIdType.MESH,
                ).wait_recv()
        pltpu.make_async_remote_copy(
            src_ref=send_buf.at[pl.ds(0, CK), :],
            dst_ref=out_ref.at[
                pl.ds(recv_block + diag_q * qm + (2 * (1 - j) + 1) * CK, CK), :
            ],
            send_sem=zsend_sems.at[4],
            recv_sem=zrecv_sems.at[4],
            device_id=zdev,
            device_id_type=pl.DeviceIdType.MESH,
        ).wait_recv()
        for r in x_rdmas:
            r.wait_send()
        for r in y_rdmas:
            r.wait_send()
        for r in z_rdmas:
            r.wait_send()

    return pl.pallas_call(
        body,
        out_shape=jax.ShapeDtypeStruct((out_m, nh), jnp.bfloat16),
        in_specs=[pl.BlockSpec(memory_space=pltpu.MemorySpace.VMEM)],
        out_specs=pl.BlockSpec(memory_space=pltpu.MemorySpace.VMEM),
        scratch_shapes=[
            pltpu.VMEM((6 * CK, nh), jnp.bfloat16),
            pltpu.SemaphoreType.DMA((6,)),
            pltpu.SemaphoreType.DMA((6,)),
            pltpu.SemaphoreType.DMA((5,)),
            pltpu.SemaphoreType.DMA((5,)),
            pltpu.SemaphoreType.DMA((5,)),
            pltpu.SemaphoreType.DMA((5,)),
        ],
        compiler_params=pltpu.CompilerParams(collective_id=0),
    )(x)
